# Initial kernel scaffold; baseline (speedup 1.0000x reference)
import jax
import jax.numpy as jnp
from jax.experimental import pallas as pl


def kernel(node_features, edge_index, W_in, b_in, W1, a_src1, a_dst1, b1, g1, be1, W2, a_src2, a_dst2, b2, g2, be2):
    raise NotImplementedError("write your pallas kernel here")


# speedup vs baseline: 64.8532x; 64.8532x over previous
import functools

import jax
import jax.numpy as jnp
from jax import lax
from jax.experimental import pallas as pl
from jax.experimental.pallas import tpu as pltpu
from jax.experimental.pallas import tpu_sc as plsc

N = 10000
E = 320000
F_IN = 128
H_DIM = 128
HEADS = 8
D_HEAD = 16

NC = 2
NS = 16
L = 16

RB = 2000
CH = 125
EPT = E // (NC * NS)
NCHUNK = EPT // CH
KB = 8
NSUPER = NCHUNK // KB
NPAD = 10240
RPS = NPAD // NS
RCH = 64
NRC = RPS // RCH


def _pre_body(nf_ref, w_ref, b_ref, x_ref):
  x_ref[...] = (
      jnp.dot(nf_ref[...], w_ref[...], preferred_element_type=jnp.float32)
      + b_ref[...]
  )


def _proj_body(x_ref, w_ref, amat_ref, h_ref, asd_ref, m_ref):
  i = pl.program_id(0)
  h = jnp.dot(x_ref[...], w_ref[...], preferred_element_type=jnp.float32)
  h_ref[...] = h
  asd = jnp.dot(h, amat_ref[...], preferred_element_type=jnp.float32)
  asd_ref[...] = asd
  bm = jnp.max(asd, axis=0, keepdims=True)

  @pl.when(i == 0)
  def _():
    m_ref[...] = jnp.full_like(m_ref, -1e30)

  m_ref[...] = jnp.maximum(m_ref[...], bm)


def _post_body(acc_ref, den_ref, res_ref, bias_ref, g_ref, be_ref, x_ref):
  a = acc_ref[0] + acc_ref[1]
  d8 = (den_ref[0] + den_ref[1])[:, 0:HEADS]
  col = lax.broadcasted_iota(jnp.int32, (HEADS, H_DIM), 1)
  row = lax.broadcasted_iota(jnp.int32, (HEADS, H_DIM), 0)
  sel = (col // D_HEAD == row).astype(jnp.float32)
  d_exp = jnp.dot(d8, sel, preferred_element_type=jnp.float32)
  out = a / jnp.maximum(d_exp, 1e-30) + bias_ref[...]
  out = jnp.where(out > 0, out, jnp.exp(jnp.minimum(out, 0.0)) - 1.0)
  xn = out + res_ref[...]
  mu = jnp.mean(xn, axis=1, keepdims=True)
  var = jnp.mean((xn - mu) * (xn - mu), axis=1, keepdims=True)
  x_ref[...] = (xn - mu) * lax.rsqrt(var + 1e-5) * g_ref[...] + be_ref[...]


def _tc_pre(nf, w_in, b_in):
  return pl.pallas_call(
      _pre_body,
      grid=(N // RB,),
      in_specs=[
          pl.BlockSpec((RB, F_IN), lambda i: (i, 0)),
          pl.BlockSpec((F_IN, H_DIM), lambda i: (0, 0)),
          pl.BlockSpec((1, H_DIM), lambda i: (0, 0)),
      ],
      out_specs=pl.BlockSpec((RB, H_DIM), lambda i: (i, 0)),
      out_shape=jax.ShapeDtypeStruct((N, H_DIM), jnp.float32),
  )(nf, w_in, b_in.reshape(1, H_DIM))


def _tc_proj(x, w, amat):
  return pl.pallas_call(
      _proj_body,
      grid=(N // RB,),
      in_specs=[
          pl.BlockSpec((RB, H_DIM), lambda i: (i, 0)),
          pl.BlockSpec((H_DIM, H_DIM), lambda i: (0, 0)),
          pl.BlockSpec((H_DIM, 32), lambda i: (0, 0)),
      ],
      out_specs=[
          pl.BlockSpec((RB, H_DIM), lambda i: (i, 0)),
          pl.BlockSpec((RB, 32), lambda i: (i, 0)),
          pl.BlockSpec((1, 32), lambda i: (0, 0)),
      ],
      out_shape=[
          jax.ShapeDtypeStruct((N, H_DIM), jnp.float32),
          jax.ShapeDtypeStruct((N, 32), jnp.float32),
          jax.ShapeDtypeStruct((1, 32), jnp.float32),
      ],
  )(x, w, amat)


def _tc_post(acc, den, res, bias, g, be):
  return pl.pallas_call(
      _post_body,
      grid=(N // RB,),
      in_specs=[
          pl.BlockSpec((NC, RB, H_DIM), lambda i: (0, i, 0)),
          pl.BlockSpec((NC, RB, 16), lambda i: (0, i, 0)),
          pl.BlockSpec((RB, H_DIM), lambda i: (i, 0)),
          pl.BlockSpec((1, H_DIM), lambda i: (0, 0)),
          pl.BlockSpec((1, H_DIM), lambda i: (0, 0)),
          pl.BlockSpec((1, H_DIM), lambda i: (0, 0)),
      ],
      out_specs=pl.BlockSpec((RB, H_DIM), lambda i: (i, 0)),
      out_shape=jax.ShapeDtypeStruct((N, H_DIM), jnp.float32),
  )(acc, den, res, bias.reshape(1, H_DIM), g.reshape(1, H_DIM),
    be.reshape(1, H_DIM))


def _edge_body(src_hbm, dst_hbm, asd_hbm, hh_hbm, m16_hbm,
               acc_hbm, den_hbm,
               acc_sh, den_sh, sidx, didx, srows, drows, wbuf, hrows,
               m16v):
  c = lax.axis_index("c")
  s = lax.axis_index("s")

  def zrow(r, _):
    for g in range(H_DIM // L):
      hrows[r, pl.ds(g * L, L)] = jnp.zeros((L,), jnp.float32)
    wbuf[r, :] = jnp.zeros((L,), jnp.float32)
    return 0

  lax.fori_loop(0, RCH, zrow, 0)
  r0 = s * RPS
  for k in range(NRC):
    pltpu.sync_copy(hrows.at[pl.ds(0, RCH)], acc_sh.at[pl.ds(r0 + k * RCH, RCH)])
    pltpu.sync_copy(wbuf.at[pl.ds(0, RCH)], den_sh.at[pl.ds(r0 + k * RCH, RCH)])
  pltpu.sync_copy(m16_hbm, m16v)
  plsc.subcore_barrier()

  m16 = m16v[...]
  tile_row0 = (c * NS + s) * NCHUNK

  def superchunk(sc, _):
    row0 = tile_row0 + sc * KB
    pltpu.sync_copy(src_hbm.at[pl.ds(row0, KB)], sidx)
    pltpu.sync_copy(dst_hbm.at[pl.ds(row0, KB)], didx)

    def chunk(j, _):
      sidx_row = sidx.at[j]
      didx_row = didx.at[j]
      pltpu.sync_copy(asd_hbm.at[sidx_row], srows)
      pltpu.sync_copy(asd_hbm.at[didx_row], drows)
      pltpu.sync_copy(hh_hbm.at[sidx_row], hrows)

      def edge_w(e, _):
        sv = srows[e, pl.ds(0, L)] + drows[e, pl.ds(L, L)]
        wbuf[e, :] = jnp.exp(jnp.maximum(sv, 0.2 * sv) - m16)
        return 0

      lax.fori_loop(0, CH, edge_w, 0)
      pltpu.sync_copy(wbuf, den_sh.at[didx_row], add=True)

      def edge_scale(e, _):
        wv = wbuf[e, :]
        for g in range(HEADS):
          hrows[e, pl.ds(g * D_HEAD, L)] = (
              hrows[e, pl.ds(g * D_HEAD, L)] * wv[g]
          )
        return 0

      lax.fori_loop(0, CH, edge_scale, 0)
      pltpu.sync_copy(hrows, acc_sh.at[didx_row], add=True)
      return 0

    lax.fori_loop(0, KB, chunk, 0)
    return 0

  lax.fori_loop(0, NSUPER, superchunk, 0)
  plsc.subcore_barrier()

  for k in range(NRC):
    rr = r0 + k * RCH
    pltpu.sync_copy(acc_sh.at[pl.ds(rr, RCH)], acc_hbm.at[c, pl.ds(rr, RCH)])
    pltpu.sync_copy(den_sh.at[pl.ds(rr, RCH)], den_hbm.at[c, pl.ds(rr, RCH)])


_edge_kernel = functools.partial(
    pl.kernel,
    out_type=[
        jax.ShapeDtypeStruct((NC, NPAD, H_DIM), jnp.float32),
        jax.ShapeDtypeStruct((NC, NPAD, 16), jnp.float32),
    ],
    mesh=plsc.VectorSubcoreMesh(
        core_axis_name="c", subcore_axis_name="s", num_cores=NC,
        num_subcores=NS),
    compiler_params=pltpu.CompilerParams(use_tc_tiling_on_sc=False),
    scratch_types=[
        pltpu.VMEM_SHARED((NPAD, H_DIM), jnp.float32),
        pltpu.VMEM_SHARED((NPAD, 16), jnp.float32),
        pltpu.VMEM((KB, CH), jnp.int32),
        pltpu.VMEM((KB, CH), jnp.int32),
        pltpu.VMEM((CH, 32), jnp.float32),
        pltpu.VMEM((CH, 32), jnp.float32),
        pltpu.VMEM((CH, 16), jnp.float32),
        pltpu.VMEM((CH, H_DIM), jnp.float32),
        pltpu.VMEM((16,), jnp.float32),
    ],
)(_edge_body)


def _sc_edge_pass(src2d, dst2d, asd, hh, m16):
  return _edge_kernel(src2d, dst2d, asd, hh, m16)


def _build_amat(a_s, a_d):
  rows = jnp.arange(H_DIM)
  amat = jnp.zeros((H_DIM, 32), jnp.float32)
  amat = amat.at[rows, rows // D_HEAD].set(a_s.reshape(-1))
  amat = amat.at[rows, 16 + rows // D_HEAD].set(a_d.reshape(-1))
  return amat


def _build_m16(mraw):
  msum = jnp.maximum(mraw[0, 0:HEADS] + mraw[0, 16:16 + HEADS], 0.0)
  return jnp.concatenate([msum, jnp.full((8,), 100.0, jnp.float32)])


@jax.jit
def kernel(node_features, edge_index, W_in, b_in, W1, a_src1, a_dst1, b1,
           g1, be1, W2, a_src2, a_dst2, b2, g2, be2):
  src2d = edge_index[0].reshape(E // CH, CH)
  dst2d = edge_index[1].reshape(E // CH, CH)

  x = _tc_pre(node_features, W_in, b_in)

  hh1, asd1, mraw1 = _tc_proj(x, W1, _build_amat(a_src1, a_dst1))
  acc1, den1 = _sc_edge_pass(src2d, dst2d, asd1, hh1, _build_m16(mraw1))
  x2 = _tc_post(acc1, den1, x, b1, g1, be1)

  hh2, asd2, mraw2 = _tc_proj(x2, W2, _build_amat(a_src2, a_dst2))
  acc2, den2 = _sc_edge_pass(src2d, dst2d, asd2, hh2, _build_m16(mraw2))
  return _tc_post(acc2, den2, x2, b2, g2, be2)


# speedup vs baseline: 86.9618x; 1.3409x over previous
import functools

import jax
import jax.numpy as jnp
from jax import lax
from jax.experimental import pallas as pl
from jax.experimental.pallas import tpu as pltpu
from jax.experimental.pallas import tpu_sc as plsc

N = 10000
E = 320000
F_IN = 128
H_DIM = 128
HEADS = 8
D_HEAD = 16

NC = 2
NS = 16
L = 16

RB = 2000
CH = 100
EPT = E // (NC * NS)
NCHUNK = EPT // CH
KB = 20
NSUPER = NCHUNK // KB
NPAD = 10240
RPS = NPAD // NS
RCH = 64
NRC = RPS // RCH


def _pre_body(nf_ref, w_ref, b_ref, x_ref):
  x_ref[...] = (
      jnp.dot(nf_ref[...], w_ref[...], preferred_element_type=jnp.float32)
      + b_ref[...]
  )


def _proj_body(x_ref, w_ref, amat_ref, h_ref, asrc_ref, adst_ref, m_ref):
  i = pl.program_id(0)
  h = jnp.dot(x_ref[...], w_ref[...], preferred_element_type=jnp.float32)
  h_ref[...] = h
  asd = jnp.dot(h, amat_ref[...], preferred_element_type=jnp.float32)
  asrc_ref[...] = asd[:, 0:16]
  adst_ref[...] = asd[:, 16:32]
  bm = jnp.max(asd, axis=0, keepdims=True)

  @pl.when(i == 0)
  def _():
    m_ref[...] = jnp.full_like(m_ref, -1e30)

  m_ref[...] = jnp.maximum(m_ref[...], bm)


def _post_body(acc_ref, den_ref, res_ref, bias_ref, g_ref, be_ref, x_ref):
  a = acc_ref[0] + acc_ref[1]
  d8 = (den_ref[0] + den_ref[1])[:, 0:HEADS]
  col = lax.broadcasted_iota(jnp.int32, (HEADS, H_DIM), 1)
  row = lax.broadcasted_iota(jnp.int32, (HEADS, H_DIM), 0)
  sel = (col // D_HEAD == row).astype(jnp.float32)
  d_exp = jnp.dot(d8, sel, preferred_element_type=jnp.float32)
  out = a / jnp.maximum(d_exp, 1e-30) + bias_ref[...]
  out = jnp.where(out > 0, out, jnp.exp(jnp.minimum(out, 0.0)) - 1.0)
  xn = out + res_ref[...]
  mu = jnp.mean(xn, axis=1, keepdims=True)
  var = jnp.mean((xn - mu) * (xn - mu), axis=1, keepdims=True)
  x_ref[...] = (xn - mu) * lax.rsqrt(var + 1e-5) * g_ref[...] + be_ref[...]


def _tc_pre(nf, w_in, b_in):
  return pl.pallas_call(
      _pre_body,
      grid=(N // RB,),
      in_specs=[
          pl.BlockSpec((RB, F_IN), lambda i: (i, 0)),
          pl.BlockSpec((F_IN, H_DIM), lambda i: (0, 0)),
          pl.BlockSpec((1, H_DIM), lambda i: (0, 0)),
      ],
      out_specs=pl.BlockSpec((RB, H_DIM), lambda i: (i, 0)),
      out_shape=jax.ShapeDtypeStruct((N, H_DIM), jnp.float32),
  )(nf, w_in, b_in.reshape(1, H_DIM))


def _tc_proj(x, w, amat):
  return pl.pallas_call(
      _proj_body,
      grid=(N // RB,),
      in_specs=[
          pl.BlockSpec((RB, H_DIM), lambda i: (i, 0)),
          pl.BlockSpec((H_DIM, H_DIM), lambda i: (0, 0)),
          pl.BlockSpec((H_DIM, 32), lambda i: (0, 0)),
      ],
      out_specs=[
          pl.BlockSpec((RB, H_DIM), lambda i: (i, 0)),
          pl.BlockSpec((RB, 16), lambda i: (i, 0)),
          pl.BlockSpec((RB, 16), lambda i: (i, 0)),
          pl.BlockSpec((1, 32), lambda i: (0, 0)),
      ],
      out_shape=[
          jax.ShapeDtypeStruct((N, H_DIM), jnp.float32),
          jax.ShapeDtypeStruct((N, 16), jnp.float32),
          jax.ShapeDtypeStruct((N, 16), jnp.float32),
          jax.ShapeDtypeStruct((1, 32), jnp.float32),
      ],
  )(x, w, amat)


def _tc_post(acc, den, res, bias, g, be):
  return pl.pallas_call(
      _post_body,
      grid=(N // RB,),
      in_specs=[
          pl.BlockSpec((NC, RB, H_DIM), lambda i: (0, i, 0)),
          pl.BlockSpec((NC, RB, 16), lambda i: (0, i, 0)),
          pl.BlockSpec((RB, H_DIM), lambda i: (i, 0)),
          pl.BlockSpec((1, H_DIM), lambda i: (0, 0)),
          pl.BlockSpec((1, H_DIM), lambda i: (0, 0)),
          pl.BlockSpec((1, H_DIM), lambda i: (0, 0)),
      ],
      out_specs=pl.BlockSpec((RB, H_DIM), lambda i: (i, 0)),
      out_shape=jax.ShapeDtypeStruct((N, H_DIM), jnp.float32),
  )(acc, den, res, bias.reshape(1, H_DIM), g.reshape(1, H_DIM),
    be.reshape(1, H_DIM))


def _edge_body(src_hbm, dst_hbm, asrc_hbm, adst_hbm, hh_hbm, m16_hbm,
               acc_hbm, den_hbm,
               acc_sh, den_sh, sidx, didx, srows, drows, wbuf,
               hrows0, hrows1,
               sem_s, sem_d, sem_h0, sem_h1, sem_den, sem_a0, sem_a1,
               m16v):
  c = lax.axis_index("c")
  s = lax.axis_index("s")

  def zrow(r, _):
    for g in range(H_DIM // L):
      hrows0[r, pl.ds(g * L, L)] = jnp.zeros((L,), jnp.float32)
    wbuf[r, :] = jnp.zeros((L,), jnp.float32)
    return 0

  lax.fori_loop(0, RCH, zrow, 0)
  r0 = s * RPS
  for k in range(NRC):
    pltpu.sync_copy(hrows0.at[pl.ds(0, RCH)],
                    acc_sh.at[pl.ds(r0 + k * RCH, RCH)])
    pltpu.sync_copy(wbuf.at[pl.ds(0, RCH)],
                    den_sh.at[pl.ds(r0 + k * RCH, RCH)])
  pltpu.sync_copy(m16_hbm, m16v)
  plsc.subcore_barrier()

  m16 = m16v[...]
  tile_row0 = (c * NS + s) * NCHUNK
  hb = (hrows0, hrows1)
  sem_h = (sem_h0, sem_h1)
  sem_a = (sem_a0, sem_a1)

  def superchunk(sc, _):
    row0 = tile_row0 + sc * KB
    pltpu.sync_copy(src_hbm.at[pl.ds(row0, KB)], sidx)
    pltpu.sync_copy(dst_hbm.at[pl.ds(row0, KB)], didx)

    d_s = pltpu.async_copy(asrc_hbm.at[sidx.at[0]], srows, sem_s)
    d_d = pltpu.async_copy(adst_hbm.at[didx.at[0]], drows, sem_d)
    d_h = pltpu.async_copy(hh_hbm.at[sidx.at[0]], hrows0, sem_h0)
    den_desc = None
    acc_desc = [None, None]

    for j in range(KB):
      p = j % 2
      d_s.wait()
      d_d.wait()
      d_h.wait()
      if den_desc is not None:
        den_desc.wait()
      hbp = hb[p]

      def edge_fused(e, _):
        sv = srows[e, :] + drows[e, :]
        w16 = jnp.exp(jnp.maximum(sv, 0.2 * sv) - m16)
        wbuf[e, :] = w16
        for g in range(HEADS):
          hbp[e, pl.ds(g * D_HEAD, L)] = hbp[e, pl.ds(g * D_HEAD, L)] * w16[g]
        return 0

      lax.fori_loop(0, CH, edge_fused, 0)

      den_desc = pltpu.async_copy(wbuf, den_sh.at[didx.at[j]], sem_den,
                                  add=True)
      acc_desc[p] = pltpu.async_copy(hbp, acc_sh.at[didx.at[j]], sem_a[p],
                                     add=True)
      if j + 1 < KB:
        d_s = pltpu.async_copy(asrc_hbm.at[sidx.at[j + 1]], srows, sem_s)
        d_d = pltpu.async_copy(adst_hbm.at[didx.at[j + 1]], drows, sem_d)
        if acc_desc[1 - p] is not None:
          acc_desc[1 - p].wait()
        d_h = pltpu.async_copy(hh_hbm.at[sidx.at[j + 1]], hb[1 - p],
                               sem_h[1 - p])

    den_desc.wait()
    acc_desc[0].wait()
    acc_desc[1].wait()
    return 0

  lax.fori_loop(0, NSUPER, superchunk, 0)
  plsc.subcore_barrier()

  for k in range(NRC):
    rr = r0 + k * RCH
    pltpu.sync_copy(acc_sh.at[pl.ds(rr, RCH)], acc_hbm.at[c, pl.ds(rr, RCH)])
    pltpu.sync_copy(den_sh.at[pl.ds(rr, RCH)], den_hbm.at[c, pl.ds(rr, RCH)])


_edge_kernel = functools.partial(
    pl.kernel,
    out_type=[
        jax.ShapeDtypeStruct((NC, NPAD, H_DIM), jnp.float32),
        jax.ShapeDtypeStruct((NC, NPAD, 16), jnp.float32),
    ],
    mesh=plsc.VectorSubcoreMesh(
        core_axis_name="c", subcore_axis_name="s", num_cores=NC,
        num_subcores=NS),
    compiler_params=pltpu.CompilerParams(use_tc_tiling_on_sc=False),
    scratch_types=[
        pltpu.VMEM_SHARED((NPAD, H_DIM), jnp.float32),
        pltpu.VMEM_SHARED((NPAD, 16), jnp.float32),
        pltpu.VMEM((KB, CH), jnp.int32),
        pltpu.VMEM((KB, CH), jnp.int32),
        pltpu.VMEM((CH, 16), jnp.float32),
        pltpu.VMEM((CH, 16), jnp.float32),
        pltpu.VMEM((CH, 16), jnp.float32),
        pltpu.VMEM((CH, H_DIM), jnp.float32),
        pltpu.VMEM((CH, H_DIM), jnp.float32),
        pltpu.SemaphoreType.DMA,
        pltpu.SemaphoreType.DMA,
        pltpu.SemaphoreType.DMA,
        pltpu.SemaphoreType.DMA,
        pltpu.SemaphoreType.DMA,
        pltpu.SemaphoreType.DMA,
        pltpu.SemaphoreType.DMA,
        pltpu.VMEM((16,), jnp.float32),
    ],
)(_edge_body)


def _sc_edge_pass(src2d, dst2d, asrc16, adst16, hh, m16):
  return _edge_kernel(src2d, dst2d, asrc16, adst16, hh, m16)


def _build_amat(a_s, a_d):
  rows = jnp.arange(H_DIM)
  amat = jnp.zeros((H_DIM, 32), jnp.float32)
  amat = amat.at[rows, rows // D_HEAD].set(a_s.reshape(-1))
  amat = amat.at[rows, 16 + rows // D_HEAD].set(a_d.reshape(-1))
  return amat


def _build_m16(mraw):
  msum = jnp.maximum(mraw[0, 0:HEADS] + mraw[0, 16:16 + HEADS], 0.0)
  return jnp.concatenate([msum, jnp.full((8,), 100.0, jnp.float32)])


@jax.jit
def kernel(node_features, edge_index, W_in, b_in, W1, a_src1, a_dst1, b1,
           g1, be1, W2, a_src2, a_dst2, b2, g2, be2):
  src2d = edge_index[0].reshape(E // CH, CH)
  dst2d = edge_index[1].reshape(E // CH, CH)

  x = _tc_pre(node_features, W_in, b_in)

  hh1, as1, ad1, mraw1 = _tc_proj(x, W1, _build_amat(a_src1, a_dst1))
  acc1, den1 = _sc_edge_pass(src2d, dst2d, as1, ad1, hh1, _build_m16(mraw1))
  x2 = _tc_post(acc1, den1, x, b1, g1, be1)

  hh2, as2, ad2, mraw2 = _tc_proj(x2, W2, _build_amat(a_src2, a_dst2))
  acc2, den2 = _sc_edge_pass(src2d, dst2d, as2, ad2, hh2, _build_m16(mraw2))
  return _tc_post(acc2, den2, x2, b2, g2, be2)


# speedup vs baseline: 137.2475x; 1.5783x over previous
import functools

import jax
import jax.numpy as jnp
from jax import lax
from jax.experimental import pallas as pl
from jax.experimental.pallas import tpu as pltpu
from jax.experimental.pallas import tpu_sc as plsc

N = 10000
E = 320000
F_IN = 128
H_DIM = 128
HEADS = 8
D_HEAD = 16

NC = 2
NS = 16
L = 16

RB = 2000
CH = 100
EPT = E // (NC * NS)
NCHUNK = EPT // CH
KB = 20
NSUPER = NCHUNK // KB
NPAD = 10240
RPS = NPAD // NS
RCH = 64
NRC = RPS // RCH


def _pre_body(nf_ref, w_ref, b_ref, x_ref):
  x_ref[...] = (
      jnp.dot(nf_ref[...], w_ref[...], preferred_element_type=jnp.float32)
      + b_ref[...]
  )


def _proj_body(x_ref, w_ref, amat_ref, h_ref, asrc_ref, adst_ref, m_ref):
  i = pl.program_id(0)
  h = jnp.dot(x_ref[...], w_ref[...], preferred_element_type=jnp.float32)
  h_ref[...] = h
  asd = jnp.dot(h, amat_ref[...], preferred_element_type=jnp.float32)
  asrc_ref[...] = asd[:, 0:16]
  adst_ref[...] = asd[:, 16:32]
  bm = jnp.max(asd, axis=0, keepdims=True)

  @pl.when(i == 0)
  def _():
    m_ref[...] = jnp.full_like(m_ref, -1e30)

  m_ref[...] = jnp.maximum(m_ref[...], bm)


def _post_body(acc_ref, den_ref, res_ref, bias_ref, g_ref, be_ref, x_ref):
  a = acc_ref[0] + acc_ref[1]
  d8 = (den_ref[0] + den_ref[1])[:, 0:HEADS]
  col = lax.broadcasted_iota(jnp.int32, (HEADS, H_DIM), 1)
  row = lax.broadcasted_iota(jnp.int32, (HEADS, H_DIM), 0)
  sel = (col // D_HEAD == row).astype(jnp.float32)
  d_exp = jnp.dot(d8, sel, preferred_element_type=jnp.float32)
  out = a / jnp.maximum(d_exp, 1e-30) + bias_ref[...]
  out = jnp.where(out > 0, out, jnp.exp(jnp.minimum(out, 0.0)) - 1.0)
  xn = out + res_ref[...]
  mu = jnp.mean(xn, axis=1, keepdims=True)
  var = jnp.mean((xn - mu) * (xn - mu), axis=1, keepdims=True)
  x_ref[...] = (xn - mu) * lax.rsqrt(var + 1e-5) * g_ref[...] + be_ref[...]


def _tc_pre(nf, w_in, b_in):
  return pl.pallas_call(
      _pre_body,
      grid=(N // RB,),
      in_specs=[
          pl.BlockSpec((RB, F_IN), lambda i: (i, 0)),
          pl.BlockSpec((F_IN, H_DIM), lambda i: (0, 0)),
          pl.BlockSpec((1, H_DIM), lambda i: (0, 0)),
      ],
      out_specs=pl.BlockSpec((RB, H_DIM), lambda i: (i, 0)),
      out_shape=jax.ShapeDtypeStruct((N, H_DIM), jnp.float32),
  )(nf, w_in, b_in.reshape(1, H_DIM))


def _tc_proj(x, w, amat):
  return pl.pallas_call(
      _proj_body,
      grid=(N // RB,),
      in_specs=[
          pl.BlockSpec((RB, H_DIM), lambda i: (i, 0)),
          pl.BlockSpec((H_DIM, H_DIM), lambda i: (0, 0)),
          pl.BlockSpec((H_DIM, 32), lambda i: (0, 0)),
      ],
      out_specs=[
          pl.BlockSpec((RB, H_DIM), lambda i: (i, 0)),
          pl.BlockSpec((RB, 16), lambda i: (i, 0)),
          pl.BlockSpec((RB, 16), lambda i: (i, 0)),
          pl.BlockSpec((1, 32), lambda i: (0, 0)),
      ],
      out_shape=[
          jax.ShapeDtypeStruct((N, H_DIM), jnp.float32),
          jax.ShapeDtypeStruct((N, 16), jnp.float32),
          jax.ShapeDtypeStruct((N, 16), jnp.float32),
          jax.ShapeDtypeStruct((1, 32), jnp.float32),
      ],
  )(x, w, amat)


def _tc_post(acc, den, res, bias, g, be):
  return pl.pallas_call(
      _post_body,
      grid=(N // RB,),
      in_specs=[
          pl.BlockSpec((NC, RB, H_DIM), lambda i: (0, i, 0)),
          pl.BlockSpec((NC, RB, 16), lambda i: (0, i, 0)),
          pl.BlockSpec((RB, H_DIM), lambda i: (i, 0)),
          pl.BlockSpec((1, H_DIM), lambda i: (0, 0)),
          pl.BlockSpec((1, H_DIM), lambda i: (0, 0)),
          pl.BlockSpec((1, H_DIM), lambda i: (0, 0)),
      ],
      out_specs=pl.BlockSpec((RB, H_DIM), lambda i: (i, 0)),
      out_shape=jax.ShapeDtypeStruct((N, H_DIM), jnp.float32),
  )(acc, den, res, bias.reshape(1, H_DIM), g.reshape(1, H_DIM),
    be.reshape(1, H_DIM))


def _edge_body(src_hbm, dst_hbm, asrc_hbm, adst_hbm, hh_hbm, m16_hbm,
               acc_hbm, den_hbm,
               acc_sh, den_sh, sidx, didx, srows, drows, wbuf,
               hrows0, hrows1,
               sem_s, sem_d, sem_h0, sem_h1, sem_den, sem_a0, sem_a1,
               m16v):
  c = lax.axis_index("c")
  s = lax.axis_index("s")

  def zrow(r, _):
    for g in range(H_DIM // L):
      hrows0[r, pl.ds(g * L, L)] = jnp.zeros((L,), jnp.float32)
    wbuf[r, :] = jnp.zeros((L,), jnp.float32)
    return 0

  lax.fori_loop(0, RCH, zrow, 0)
  r0 = s * RPS
  for k in range(NRC):
    pltpu.sync_copy(hrows0.at[pl.ds(0, RCH)],
                    acc_sh.at[pl.ds(r0 + k * RCH, RCH)])
    pltpu.sync_copy(wbuf.at[pl.ds(0, RCH)],
                    den_sh.at[pl.ds(r0 + k * RCH, RCH)])
  pltpu.sync_copy(m16_hbm, m16v)
  plsc.subcore_barrier()

  m16 = m16v[...]
  tile_row0 = (c * NS + s) * NCHUNK
  hb = (hrows0, hrows1)
  sem_h = (sem_h0, sem_h1)
  sem_a = (sem_a0, sem_a1)

  def superchunk(sc, _):
    row0 = tile_row0 + sc * KB
    pltpu.sync_copy(src_hbm.at[pl.ds(row0, KB)], sidx)
    pltpu.sync_copy(dst_hbm.at[pl.ds(row0, KB)], didx)

    d_s = pltpu.async_copy(asrc_hbm.at[sidx.at[0]], srows, sem_s)
    d_d = pltpu.async_copy(adst_hbm.at[didx.at[0]], drows, sem_d)
    d_h = pltpu.async_copy(hh_hbm.at[sidx.at[0]], hrows0, sem_h0)
    den_desc = None
    acc_desc = [None, None]

    for j in range(KB):
      p = j % 2
      d_s.wait()
      d_d.wait()
      d_h.wait()
      if den_desc is not None:
        den_desc.wait()
      hbp = hb[p]

      @plsc.parallel_loop(0, CH, unroll=4)
      def edge_fused(e):
        sv = srows[e, :] + drows[e, :]
        w16 = jnp.exp(jnp.maximum(sv, 0.2 * sv) - m16)
        wbuf[e, :] = w16
        for g in range(HEADS):
          hbp[e, pl.ds(g * D_HEAD, L)] = hbp[e, pl.ds(g * D_HEAD, L)] * w16[g]

      den_desc = pltpu.async_copy(wbuf, den_sh.at[didx.at[j]], sem_den,
                                  add=True)
      acc_desc[p] = pltpu.async_copy(hbp, acc_sh.at[didx.at[j]], sem_a[p],
                                     add=True)
      if j + 1 < KB:
        d_s = pltpu.async_copy(asrc_hbm.at[sidx.at[j + 1]], srows, sem_s)
        d_d = pltpu.async_copy(adst_hbm.at[didx.at[j + 1]], drows, sem_d)
        if acc_desc[1 - p] is not None:
          acc_desc[1 - p].wait()
        d_h = pltpu.async_copy(hh_hbm.at[sidx.at[j + 1]], hb[1 - p],
                               sem_h[1 - p])

    den_desc.wait()
    acc_desc[0].wait()
    acc_desc[1].wait()
    return 0

  lax.fori_loop(0, NSUPER, superchunk, 0)
  plsc.subcore_barrier()

  for k in range(NRC):
    rr = r0 + k * RCH
    pltpu.sync_copy(acc_sh.at[pl.ds(rr, RCH)], acc_hbm.at[c, pl.ds(rr, RCH)])
    pltpu.sync_copy(den_sh.at[pl.ds(rr, RCH)], den_hbm.at[c, pl.ds(rr, RCH)])


_edge_kernel = functools.partial(
    pl.kernel,
    out_type=[
        jax.ShapeDtypeStruct((NC, NPAD, H_DIM), jnp.float32),
        jax.ShapeDtypeStruct((NC, NPAD, 16), jnp.float32),
    ],
    mesh=plsc.VectorSubcoreMesh(
        core_axis_name="c", subcore_axis_name="s", num_cores=NC,
        num_subcores=NS),
    compiler_params=pltpu.CompilerParams(use_tc_tiling_on_sc=False),
    scratch_types=[
        pltpu.VMEM_SHARED((NPAD, H_DIM), jnp.float32),
        pltpu.VMEM_SHARED((NPAD, 16), jnp.float32),
        pltpu.VMEM((KB, CH), jnp.int32),
        pltpu.VMEM((KB, CH), jnp.int32),
        pltpu.VMEM((CH, 16), jnp.float32),
        pltpu.VMEM((CH, 16), jnp.float32),
        pltpu.VMEM((CH, 16), jnp.float32),
        pltpu.VMEM((CH, H_DIM), jnp.float32),
        pltpu.VMEM((CH, H_DIM), jnp.float32),
        pltpu.SemaphoreType.DMA,
        pltpu.SemaphoreType.DMA,
        pltpu.SemaphoreType.DMA,
        pltpu.SemaphoreType.DMA,
        pltpu.SemaphoreType.DMA,
        pltpu.SemaphoreType.DMA,
        pltpu.SemaphoreType.DMA,
        pltpu.VMEM((16,), jnp.float32),
    ],
)(_edge_body)


def _sc_edge_pass(src2d, dst2d, asrc16, adst16, hh, m16):
  return _edge_kernel(src2d, dst2d, asrc16, adst16, hh, m16)


def _build_amat(a_s, a_d):
  rows = jnp.arange(H_DIM)
  amat = jnp.zeros((H_DIM, 32), jnp.float32)
  amat = amat.at[rows, rows // D_HEAD].set(a_s.reshape(-1))
  amat = amat.at[rows, 16 + rows // D_HEAD].set(a_d.reshape(-1))
  return amat


def _build_m16(mraw):
  msum = jnp.maximum(mraw[0, 0:HEADS] + mraw[0, 16:16 + HEADS], 0.0)
  return jnp.concatenate([msum, jnp.full((8,), 100.0, jnp.float32)])


@jax.jit
def kernel(node_features, edge_index, W_in, b_in, W1, a_src1, a_dst1, b1,
           g1, be1, W2, a_src2, a_dst2, b2, g2, be2):
  src2d = edge_index[0].reshape(E // CH, CH)
  dst2d = edge_index[1].reshape(E // CH, CH)

  x = _tc_pre(node_features, W_in, b_in)

  hh1, as1, ad1, mraw1 = _tc_proj(x, W1, _build_amat(a_src1, a_dst1))
  acc1, den1 = _sc_edge_pass(src2d, dst2d, as1, ad1, hh1, _build_m16(mraw1))
  x2 = _tc_post(acc1, den1, x, b1, g1, be1)

  hh2, as2, ad2, mraw2 = _tc_proj(x2, W2, _build_amat(a_src2, a_dst2))
  acc2, den2 = _sc_edge_pass(src2d, dst2d, as2, ad2, hh2, _build_m16(mraw2))
  return _tc_post(acc2, den2, x2, b2, g2, be2)


# speedup vs baseline: 140.8295x; 1.0261x over previous
import functools

import jax
import jax.numpy as jnp
from jax import lax
from jax.experimental import pallas as pl
from jax.experimental.pallas import tpu as pltpu
from jax.experimental.pallas import tpu_sc as plsc

N = 10000
E = 320000
F_IN = 128
H_DIM = 128
HEADS = 8
D_HEAD = 16

NC = 2
NS = 16
L = 16

RB = 2000
CH = 100
EPT = E // (NC * NS)
NCHUNK = EPT // CH
KB = 20
NSUPER = NCHUNK // KB
NPAD = 10240
RPS = NPAD // NS
RCH = 64
NRC = RPS // RCH


def _pre_body(nf_ref, w_ref, b_ref, x_ref):
  x_ref[...] = (
      jnp.dot(nf_ref[...], w_ref[...], preferred_element_type=jnp.float32)
      + b_ref[...]
  )


def _proj_body(x_ref, w_ref, amat_ref, h_ref, asrc_ref, adst_ref, m_ref):
  i = pl.program_id(0)
  h = jnp.dot(x_ref[...], w_ref[...], preferred_element_type=jnp.float32)
  h_ref[...] = h
  asd = jnp.dot(h, amat_ref[...], preferred_element_type=jnp.float32)
  asrc_ref[...] = asd[:, 0:16]
  adst_ref[...] = asd[:, 16:32]
  bm = jnp.max(asd, axis=0, keepdims=True)

  @pl.when(i == 0)
  def _():
    m_ref[...] = jnp.full_like(m_ref, -1e30)

  m_ref[...] = jnp.maximum(m_ref[...], bm)


def _post_body(acc_ref, den_ref, res_ref, bias_ref, g_ref, be_ref, x_ref):
  a = acc_ref[0] + acc_ref[1]
  d8 = (den_ref[0] + den_ref[1])[:, 0:HEADS]
  col = lax.broadcasted_iota(jnp.int32, (HEADS, H_DIM), 1)
  row = lax.broadcasted_iota(jnp.int32, (HEADS, H_DIM), 0)
  sel = (col // D_HEAD == row).astype(jnp.float32)
  d_exp = jnp.dot(d8, sel, preferred_element_type=jnp.float32)
  out = a / jnp.maximum(d_exp, 1e-30) + bias_ref[...]
  out = jnp.where(out > 0, out, jnp.exp(jnp.minimum(out, 0.0)) - 1.0)
  xn = out + res_ref[...]
  mu = jnp.mean(xn, axis=1, keepdims=True)
  var = jnp.mean((xn - mu) * (xn - mu), axis=1, keepdims=True)
  x_ref[...] = (xn - mu) * lax.rsqrt(var + 1e-5) * g_ref[...] + be_ref[...]


def _tc_pre(nf, w_in, b_in):
  return pl.pallas_call(
      _pre_body,
      grid=(N // RB,),
      in_specs=[
          pl.BlockSpec((RB, F_IN), lambda i: (i, 0)),
          pl.BlockSpec((F_IN, H_DIM), lambda i: (0, 0)),
          pl.BlockSpec((1, H_DIM), lambda i: (0, 0)),
      ],
      out_specs=pl.BlockSpec((RB, H_DIM), lambda i: (i, 0)),
      out_shape=jax.ShapeDtypeStruct((N, H_DIM), jnp.float32),
  )(nf, w_in, b_in.reshape(1, H_DIM))


def _tc_proj(x, w, amat):
  return pl.pallas_call(
      _proj_body,
      grid=(N // RB,),
      in_specs=[
          pl.BlockSpec((RB, H_DIM), lambda i: (i, 0)),
          pl.BlockSpec((H_DIM, H_DIM), lambda i: (0, 0)),
          pl.BlockSpec((H_DIM, 32), lambda i: (0, 0)),
      ],
      out_specs=[
          pl.BlockSpec((RB, H_DIM), lambda i: (i, 0)),
          pl.BlockSpec((RB, 16), lambda i: (i, 0)),
          pl.BlockSpec((RB, 16), lambda i: (i, 0)),
          pl.BlockSpec((1, 32), lambda i: (0, 0)),
      ],
      out_shape=[
          jax.ShapeDtypeStruct((N, H_DIM), jnp.float32),
          jax.ShapeDtypeStruct((N, 16), jnp.float32),
          jax.ShapeDtypeStruct((N, 16), jnp.float32),
          jax.ShapeDtypeStruct((1, 32), jnp.float32),
      ],
  )(x, w, amat)


def _tc_post(acc, den, res, bias, g, be):
  return pl.pallas_call(
      _post_body,
      grid=(N // RB,),
      in_specs=[
          pl.BlockSpec((NC, RB, H_DIM), lambda i: (0, i, 0)),
          pl.BlockSpec((NC, RB, 16), lambda i: (0, i, 0)),
          pl.BlockSpec((RB, H_DIM), lambda i: (i, 0)),
          pl.BlockSpec((1, H_DIM), lambda i: (0, 0)),
          pl.BlockSpec((1, H_DIM), lambda i: (0, 0)),
          pl.BlockSpec((1, H_DIM), lambda i: (0, 0)),
      ],
      out_specs=pl.BlockSpec((RB, H_DIM), lambda i: (i, 0)),
      out_shape=jax.ShapeDtypeStruct((N, H_DIM), jnp.float32),
  )(acc, den, res, bias.reshape(1, H_DIM), g.reshape(1, H_DIM),
    be.reshape(1, H_DIM))


def _edge_body(src_hbm, dst_hbm, asrc_hbm, adst_hbm, hh_hbm, mraw_hbm,
               acc_hbm, den_hbm,
               acc_sh, den_sh, sidx, didx, srows, drows, wbuf,
               hrows0, hrows1,
               sem_s, sem_d, sem_h0, sem_h1, sem_den, sem_a0, sem_a1,
               m16v):
  c = lax.axis_index("c")
  s = lax.axis_index("s")

  def zrow(r, _):
    for g in range(H_DIM // L):
      hrows0[r, pl.ds(g * L, L)] = jnp.zeros((L,), jnp.float32)
    wbuf[r, :] = jnp.zeros((L,), jnp.float32)
    return 0

  lax.fori_loop(0, RCH, zrow, 0)
  r0 = s * RPS
  zdescs = []
  for k in range(NRC):
    zdescs.append(pltpu.async_copy(
        hrows0.at[pl.ds(0, RCH)], acc_sh.at[pl.ds(r0 + k * RCH, RCH)],
        sem_a0))
    zdescs.append(pltpu.async_copy(
        wbuf.at[pl.ds(0, RCH)], den_sh.at[pl.ds(r0 + k * RCH, RCH)],
        sem_den))
  for d in zdescs:
    d.wait()
  pltpu.sync_copy(mraw_hbm, m16v)
  plsc.subcore_barrier()

  iota = lax.iota(jnp.int32, L)
  msum = jnp.maximum(m16v[pl.ds(0, L)] + m16v[pl.ds(L, L)], 0.0)
  m16 = msum + jnp.where(iota < HEADS, 0.0, 100.0)
  tile_row0 = (c * NS + s) * NCHUNK
  hb = (hrows0, hrows1)
  sem_h = (sem_h0, sem_h1)
  sem_a = (sem_a0, sem_a1)

  def superchunk(sc, _):
    row0 = tile_row0 + sc * KB
    pltpu.sync_copy(src_hbm.at[pl.ds(row0, KB)], sidx)
    pltpu.sync_copy(dst_hbm.at[pl.ds(row0, KB)], didx)

    d_s = pltpu.async_copy(asrc_hbm.at[sidx.at[0]], srows, sem_s)
    d_d = pltpu.async_copy(adst_hbm.at[didx.at[0]], drows, sem_d)
    d_h = pltpu.async_copy(hh_hbm.at[sidx.at[0]], hrows0, sem_h0)
    den_desc = None
    acc_desc = [None, None]

    for j in range(KB):
      p = j % 2
      d_s.wait()
      d_d.wait()
      d_h.wait()
      if den_desc is not None:
        den_desc.wait()
      hbp = hb[p]

      @plsc.parallel_loop(0, CH, unroll=4)
      def edge_fused(e):
        sv = srows[e, :] + drows[e, :]
        w16 = jnp.exp(jnp.maximum(sv, 0.2 * sv) - m16)
        wbuf[e, :] = w16
        for g in range(HEADS):
          hbp[e, pl.ds(g * D_HEAD, L)] = hbp[e, pl.ds(g * D_HEAD, L)] * w16[g]

      den_desc = pltpu.async_copy(wbuf, den_sh.at[didx.at[j]], sem_den,
                                  add=True)
      acc_desc[p] = pltpu.async_copy(hbp, acc_sh.at[didx.at[j]], sem_a[p],
                                     add=True)
      if j + 1 < KB:
        d_s = pltpu.async_copy(asrc_hbm.at[sidx.at[j + 1]], srows, sem_s)
        d_d = pltpu.async_copy(adst_hbm.at[didx.at[j + 1]], drows, sem_d)
        if acc_desc[1 - p] is not None:
          acc_desc[1 - p].wait()
        d_h = pltpu.async_copy(hh_hbm.at[sidx.at[j + 1]], hb[1 - p],
                               sem_h[1 - p])

    den_desc.wait()
    acc_desc[0].wait()
    acc_desc[1].wait()
    return 0

  lax.fori_loop(0, NSUPER, superchunk, 0)
  plsc.subcore_barrier()

  wdescs = []
  for k in range(NRC):
    rr = r0 + k * RCH
    wdescs.append(pltpu.async_copy(
        acc_sh.at[pl.ds(rr, RCH)], acc_hbm.at[c, pl.ds(rr, RCH)], sem_a0))
    wdescs.append(pltpu.async_copy(
        den_sh.at[pl.ds(rr, RCH)], den_hbm.at[c, pl.ds(rr, RCH)], sem_den))
  for d in wdescs:
    d.wait()


_edge_kernel = functools.partial(
    pl.kernel,
    out_type=[
        jax.ShapeDtypeStruct((NC, NPAD, H_DIM), jnp.float32),
        jax.ShapeDtypeStruct((NC, NPAD, 16), jnp.float32),
    ],
    mesh=plsc.VectorSubcoreMesh(
        core_axis_name="c", subcore_axis_name="s", num_cores=NC,
        num_subcores=NS),
    compiler_params=pltpu.CompilerParams(use_tc_tiling_on_sc=False),
    scratch_types=[
        pltpu.VMEM_SHARED((NPAD, H_DIM), jnp.float32),
        pltpu.VMEM_SHARED((NPAD, 16), jnp.float32),
        pltpu.VMEM((KB, CH), jnp.int32),
        pltpu.VMEM((KB, CH), jnp.int32),
        pltpu.VMEM((CH, 16), jnp.float32),
        pltpu.VMEM((CH, 16), jnp.float32),
        pltpu.VMEM((CH, 16), jnp.float32),
        pltpu.VMEM((CH, H_DIM), jnp.float32),
        pltpu.VMEM((CH, H_DIM), jnp.float32),
        pltpu.SemaphoreType.DMA,
        pltpu.SemaphoreType.DMA,
        pltpu.SemaphoreType.DMA,
        pltpu.SemaphoreType.DMA,
        pltpu.SemaphoreType.DMA,
        pltpu.SemaphoreType.DMA,
        pltpu.SemaphoreType.DMA,
        pltpu.VMEM((32,), jnp.float32),
    ],
)(_edge_body)


def _sc_edge_pass(src2d, dst2d, asrc16, adst16, hh, mraw):
  return _edge_kernel(src2d, dst2d, asrc16, adst16, hh, mraw.reshape(32))


def _build_amat(a_s, a_d):
  rows = jnp.arange(H_DIM)
  amat = jnp.zeros((H_DIM, 32), jnp.float32)
  amat = amat.at[rows, rows // D_HEAD].set(a_s.reshape(-1))
  amat = amat.at[rows, 16 + rows // D_HEAD].set(a_d.reshape(-1))
  return amat


@jax.jit
def kernel(node_features, edge_index, W_in, b_in, W1, a_src1, a_dst1, b1,
           g1, be1, W2, a_src2, a_dst2, b2, g2, be2):
  src2d = edge_index[0].reshape(E // CH, CH)
  dst2d = edge_index[1].reshape(E // CH, CH)

  x = _tc_pre(node_features, W_in, b_in)

  hh1, as1, ad1, mraw1 = _tc_proj(x, W1, _build_amat(a_src1, a_dst1))
  acc1, den1 = _sc_edge_pass(src2d, dst2d, as1, ad1, hh1, mraw1)
  x2 = _tc_post(acc1, den1, x, b1, g1, be1)

  hh2, as2, ad2, mraw2 = _tc_proj(x2, W2, _build_amat(a_src2, a_dst2))
  acc2, den2 = _sc_edge_pass(src2d, dst2d, as2, ad2, hh2, mraw2)
  return _tc_post(acc2, den2, x2, b2, g2, be2)


# speedup vs baseline: 144.7857x; 1.0281x over previous
import functools

import jax
import jax.numpy as jnp
from jax import lax
from jax.experimental import pallas as pl
from jax.experimental.pallas import tpu as pltpu
from jax.experimental.pallas import tpu_sc as plsc

N = 10000
E = 320000
F_IN = 128
H_DIM = 128
HEADS = 8
D_HEAD = 16

NC = 2
NS = 16
L = 16

RB = 2000
CH = 80
EPT = E // (NC * NS)
NCHUNK = EPT // CH
KB = 5
NSUPER = NCHUNK // KB
NPAD = 10240
RPS = NPAD // NS
RCH = 64
NRC = RPS // RCH


def _pre_body(nf_ref, w_ref, b_ref, x_ref):
  x_ref[...] = (
      jnp.dot(nf_ref[...], w_ref[...], preferred_element_type=jnp.float32)
      + b_ref[...]
  )


def _proj_body(x_ref, w_ref, amat_ref, h_ref, asrc_ref, adst_ref, m_ref):
  i = pl.program_id(0)
  h = jnp.dot(x_ref[...], w_ref[...], preferred_element_type=jnp.float32)
  h_ref[...] = h
  asd = jnp.dot(h, amat_ref[...], preferred_element_type=jnp.float32)
  asrc_ref[...] = asd[:, 0:16]
  adst_ref[...] = asd[:, 16:32]
  bm = jnp.max(asd, axis=0, keepdims=True)

  @pl.when(i == 0)
  def _():
    m_ref[...] = jnp.full_like(m_ref, -1e30)

  m_ref[...] = jnp.maximum(m_ref[...], bm)


def _post_body(acc_ref, den_ref, res_ref, bias_ref, g_ref, be_ref, x_ref):
  a = acc_ref[0] + acc_ref[1]
  d8 = (den_ref[0] + den_ref[1])[:, 0:HEADS]
  col = lax.broadcasted_iota(jnp.int32, (HEADS, H_DIM), 1)
  row = lax.broadcasted_iota(jnp.int32, (HEADS, H_DIM), 0)
  sel = (col // D_HEAD == row).astype(jnp.float32)
  d_exp = jnp.dot(d8, sel, preferred_element_type=jnp.float32)
  out = a / jnp.maximum(d_exp, 1e-30) + bias_ref[...]
  out = jnp.where(out > 0, out, jnp.exp(jnp.minimum(out, 0.0)) - 1.0)
  xn = out + res_ref[...]
  mu = jnp.mean(xn, axis=1, keepdims=True)
  var = jnp.mean((xn - mu) * (xn - mu), axis=1, keepdims=True)
  x_ref[...] = (xn - mu) * lax.rsqrt(var + 1e-5) * g_ref[...] + be_ref[...]


def _tc_pre(nf, w_in, b_in):
  return pl.pallas_call(
      _pre_body,
      grid=(N // RB,),
      in_specs=[
          pl.BlockSpec((RB, F_IN), lambda i: (i, 0)),
          pl.BlockSpec((F_IN, H_DIM), lambda i: (0, 0)),
          pl.BlockSpec((1, H_DIM), lambda i: (0, 0)),
      ],
      out_specs=pl.BlockSpec((RB, H_DIM), lambda i: (i, 0)),
      out_shape=jax.ShapeDtypeStruct((N, H_DIM), jnp.float32),
  )(nf, w_in, b_in.reshape(1, H_DIM))


def _tc_proj(x, w, amat):
  return pl.pallas_call(
      _proj_body,
      grid=(N // RB,),
      in_specs=[
          pl.BlockSpec((RB, H_DIM), lambda i: (i, 0)),
          pl.BlockSpec((H_DIM, H_DIM), lambda i: (0, 0)),
          pl.BlockSpec((H_DIM, 32), lambda i: (0, 0)),
      ],
      out_specs=[
          pl.BlockSpec((RB, H_DIM), lambda i: (i, 0)),
          pl.BlockSpec((RB, 16), lambda i: (i, 0)),
          pl.BlockSpec((RB, 16), lambda i: (i, 0)),
          pl.BlockSpec((1, 32), lambda i: (0, 0)),
      ],
      out_shape=[
          jax.ShapeDtypeStruct((N, H_DIM), jnp.float32),
          jax.ShapeDtypeStruct((N, 16), jnp.float32),
          jax.ShapeDtypeStruct((N, 16), jnp.float32),
          jax.ShapeDtypeStruct((1, 32), jnp.float32),
      ],
  )(x, w, amat)


def _tc_post(acc, den, res, bias, g, be):
  return pl.pallas_call(
      _post_body,
      grid=(N // RB,),
      in_specs=[
          pl.BlockSpec((NC, RB, H_DIM), lambda i: (0, i, 0)),
          pl.BlockSpec((NC, RB, 16), lambda i: (0, i, 0)),
          pl.BlockSpec((RB, H_DIM), lambda i: (i, 0)),
          pl.BlockSpec((1, H_DIM), lambda i: (0, 0)),
          pl.BlockSpec((1, H_DIM), lambda i: (0, 0)),
          pl.BlockSpec((1, H_DIM), lambda i: (0, 0)),
      ],
      out_specs=pl.BlockSpec((RB, H_DIM), lambda i: (i, 0)),
      out_shape=jax.ShapeDtypeStruct((N, H_DIM), jnp.float32),
  )(acc, den, res, bias.reshape(1, H_DIM), g.reshape(1, H_DIM),
    be.reshape(1, H_DIM))


def _edge_body(src_hbm, dst_hbm, asrc_hbm, adst_hbm, hh_hbm, mraw_hbm,
               acc_hbm, den_hbm,
               acc_sh, den_sh, sidx, didx, srows, drows, wbuf,
               hrows0, hrows1, hrows2,
               sem_s, sem_d, sem_h0, sem_h1, sem_h2, sem_den,
               sem_a0, sem_a1, sem_a2,
               m16v):
  c = lax.axis_index("c")
  s = lax.axis_index("s")

  def zrow(r, _):
    for g in range(H_DIM // L):
      hrows0[r, pl.ds(g * L, L)] = jnp.zeros((L,), jnp.float32)
    wbuf[r, :] = jnp.zeros((L,), jnp.float32)
    return 0

  lax.fori_loop(0, RCH, zrow, 0)
  r0 = s * RPS
  zdescs = []
  for k in range(NRC):
    zdescs.append(pltpu.async_copy(
        hrows0.at[pl.ds(0, RCH)], acc_sh.at[pl.ds(r0 + k * RCH, RCH)],
        sem_a0))
    zdescs.append(pltpu.async_copy(
        wbuf.at[pl.ds(0, RCH)], den_sh.at[pl.ds(r0 + k * RCH, RCH)],
        sem_den))
  for d in zdescs:
    d.wait()
  pltpu.sync_copy(mraw_hbm, m16v)
  plsc.subcore_barrier()

  iota = lax.iota(jnp.int32, L)
  msum = jnp.maximum(m16v[pl.ds(0, L)] + m16v[pl.ds(L, L)], 0.0)
  m16 = msum + jnp.where(iota < HEADS, 0.0, 100.0)
  tile_row0 = (c * NS + s) * NCHUNK
  hb = (hrows0, hrows1, hrows2)
  sem_h = (sem_h0, sem_h1, sem_h2)
  sem_a = (sem_a0, sem_a1, sem_a2)

  def superchunk(sc, _):
    row0 = tile_row0 + sc * KB
    pltpu.sync_copy(src_hbm.at[pl.ds(row0, KB)], sidx)
    pltpu.sync_copy(dst_hbm.at[pl.ds(row0, KB)], didx)

    d_s = pltpu.async_copy(asrc_hbm.at[sidx.at[0]], srows, sem_s)
    d_d = pltpu.async_copy(adst_hbm.at[didx.at[0]], drows, sem_d)
    d_h = [None, None, None]
    d_h[0] = pltpu.async_copy(hh_hbm.at[sidx.at[0]], hrows0, sem_h0)
    den_desc = None
    acc_desc = [None, None, None]

    for j in range(KB):
      p = j % 3
      q = (j + 1) % 3
      d_s.wait()
      d_d.wait()
      d_h[p].wait()
      if den_desc is not None:
        den_desc.wait()
      if j + 1 < KB:
        if acc_desc[q] is not None:
          acc_desc[q].wait()
        d_h[q] = pltpu.async_copy(hh_hbm.at[sidx.at[j + 1]], hb[q], sem_h[q])
      hbp = hb[p]

      @plsc.parallel_loop(0, CH, unroll=4)
      def edge_fused(e):
        sv = srows[e, :] + drows[e, :]
        w16 = jnp.exp(jnp.maximum(sv, 0.2 * sv) - m16)
        wbuf[e, :] = w16
        for g in range(HEADS):
          hbp[e, pl.ds(g * D_HEAD, L)] = hbp[e, pl.ds(g * D_HEAD, L)] * w16[g]

      den_desc = pltpu.async_copy(wbuf, den_sh.at[didx.at[j]], sem_den,
                                  add=True)
      acc_desc[p] = pltpu.async_copy(hbp, acc_sh.at[didx.at[j]], sem_a[p],
                                     add=True)
      if j + 1 < KB:
        d_s = pltpu.async_copy(asrc_hbm.at[sidx.at[j + 1]], srows, sem_s)
        d_d = pltpu.async_copy(adst_hbm.at[didx.at[j + 1]], drows, sem_d)

    den_desc.wait()
    for d in acc_desc:
      if d is not None:
        d.wait()
    return 0

  lax.fori_loop(0, NSUPER, superchunk, 0)
  plsc.subcore_barrier()

  wdescs = []
  for k in range(NRC):
    rr = r0 + k * RCH
    wdescs.append(pltpu.async_copy(
        acc_sh.at[pl.ds(rr, RCH)], acc_hbm.at[c, pl.ds(rr, RCH)], sem_a0))
    wdescs.append(pltpu.async_copy(
        den_sh.at[pl.ds(rr, RCH)], den_hbm.at[c, pl.ds(rr, RCH)], sem_den))
  for d in wdescs:
    d.wait()


_edge_kernel = functools.partial(
    pl.kernel,
    out_type=[
        jax.ShapeDtypeStruct((NC, NPAD, H_DIM), jnp.float32),
        jax.ShapeDtypeStruct((NC, NPAD, 16), jnp.float32),
    ],
    mesh=plsc.VectorSubcoreMesh(
        core_axis_name="c", subcore_axis_name="s", num_cores=NC,
        num_subcores=NS),
    compiler_params=pltpu.CompilerParams(use_tc_tiling_on_sc=False),
    scratch_types=[
        pltpu.VMEM_SHARED((NPAD, H_DIM), jnp.float32),
        pltpu.VMEM_SHARED((NPAD, 16), jnp.float32),
        pltpu.VMEM((KB, CH), jnp.int32),
        pltpu.VMEM((KB, CH), jnp.int32),
        pltpu.VMEM((CH, 16), jnp.float32),
        pltpu.VMEM((CH, 16), jnp.float32),
        pltpu.VMEM((CH, 16), jnp.float32),
        pltpu.VMEM((CH, H_DIM), jnp.float32),
        pltpu.VMEM((CH, H_DIM), jnp.float32),
        pltpu.VMEM((CH, H_DIM), jnp.float32),
        pltpu.SemaphoreType.DMA,
        pltpu.SemaphoreType.DMA,
        pltpu.SemaphoreType.DMA,
        pltpu.SemaphoreType.DMA,
        pltpu.SemaphoreType.DMA,
        pltpu.SemaphoreType.DMA,
        pltpu.SemaphoreType.DMA,
        pltpu.SemaphoreType.DMA,
        pltpu.SemaphoreType.DMA,
        pltpu.VMEM((32,), jnp.float32),
    ],
)(_edge_body)


def _sc_edge_pass(src2d, dst2d, asrc16, adst16, hh, mraw):
  return _edge_kernel(src2d, dst2d, asrc16, adst16, hh, mraw.reshape(32))


def _build_amat(a_s, a_d):
  rows = jnp.arange(H_DIM)
  amat = jnp.zeros((H_DIM, 32), jnp.float32)
  amat = amat.at[rows, rows // D_HEAD].set(a_s.reshape(-1))
  amat = amat.at[rows, 16 + rows // D_HEAD].set(a_d.reshape(-1))
  return amat


@jax.jit
def kernel(node_features, edge_index, W_in, b_in, W1, a_src1, a_dst1, b1,
           g1, be1, W2, a_src2, a_dst2, b2, g2, be2):
  src2d = edge_index[0].reshape(E // CH, CH)
  dst2d = edge_index[1].reshape(E // CH, CH)

  x = _tc_pre(node_features, W_in, b_in)

  hh1, as1, ad1, mraw1 = _tc_proj(x, W1, _build_amat(a_src1, a_dst1))
  acc1, den1 = _sc_edge_pass(src2d, dst2d, as1, ad1, hh1, mraw1)
  x2 = _tc_post(acc1, den1, x, b1, g1, be1)

  hh2, as2, ad2, mraw2 = _tc_proj(x2, W2, _build_amat(a_src2, a_dst2))
  acc2, den2 = _sc_edge_pass(src2d, dst2d, as2, ad2, hh2, mraw2)
  return _tc_post(acc2, den2, x2, b2, g2, be2)


# speedup vs baseline: 166.5548x; 1.1504x over previous
import functools

import jax
import jax.numpy as jnp
from jax import lax
from jax.experimental import pallas as pl
from jax.experimental.pallas import tpu as pltpu
from jax.experimental.pallas import tpu_sc as plsc

N = 10000
E = 320000
F_IN = 128
H_DIM = 128
HEADS = 8
D_HEAD = 16

NC = 2
NS = 16
L = 16

RB = 2000
CH = 80
EPT = E // (NC * NS)
NCHUNK = EPT // CH
KB = 25
NSUPER = NCHUNK // KB
NPAD = 10080
RPS = NPAD // NS
RCH = 63
NRC = RPS // RCH


def _pre_body(nf_ref, w_ref, b_ref, x_ref):
  x_ref[...] = (
      jnp.dot(nf_ref[...], w_ref[...], preferred_element_type=jnp.float32)
      + b_ref[...]
  )


def _proj_body(x_ref, w_ref, amat_ref, h_ref, asrc_ref, adst_ref, m_ref):
  i = pl.program_id(0)
  h = jnp.dot(x_ref[...], w_ref[...], preferred_element_type=jnp.float32)
  h_ref[...] = h
  asd = jnp.dot(h, amat_ref[...], preferred_element_type=jnp.float32)
  asrc_ref[...] = asd[:, 0:16]
  adst_ref[...] = asd[:, 16:32]
  bm = jnp.max(asd, axis=0, keepdims=True)

  @pl.when(i == 0)
  def _():
    m_ref[...] = jnp.full_like(m_ref, -1e30)

  m_ref[...] = jnp.maximum(m_ref[...], bm)


def _post_body(acc_ref, den_ref, res_ref, bias_ref, g_ref, be_ref, x_ref):
  a = acc_ref[0] + acc_ref[1]
  d8 = (den_ref[0] + den_ref[1])[:, 0:HEADS]
  col = lax.broadcasted_iota(jnp.int32, (HEADS, H_DIM), 1)
  row = lax.broadcasted_iota(jnp.int32, (HEADS, H_DIM), 0)
  sel = (col // D_HEAD == row).astype(jnp.float32)
  d_exp = jnp.dot(d8, sel, preferred_element_type=jnp.float32)
  out = a / jnp.maximum(d_exp, 1e-30) + bias_ref[...]
  out = jnp.where(out > 0, out, jnp.exp(jnp.minimum(out, 0.0)) - 1.0)
  xn = out + res_ref[...]
  mu = jnp.mean(xn, axis=1, keepdims=True)
  var = jnp.mean((xn - mu) * (xn - mu), axis=1, keepdims=True)
  x_ref[...] = (xn - mu) * lax.rsqrt(var + 1e-5) * g_ref[...] + be_ref[...]


def _tc_pre(nf, w_in, b_in):
  return pl.pallas_call(
      _pre_body,
      grid=(N // RB,),
      in_specs=[
          pl.BlockSpec((RB, F_IN), lambda i: (i, 0)),
          pl.BlockSpec((F_IN, H_DIM), lambda i: (0, 0)),
          pl.BlockSpec((1, H_DIM), lambda i: (0, 0)),
      ],
      out_specs=pl.BlockSpec((RB, H_DIM), lambda i: (i, 0)),
      out_shape=jax.ShapeDtypeStruct((N, H_DIM), jnp.float32),
  )(nf, w_in, b_in.reshape(1, H_DIM))


def _tc_proj(x, w, amat):
  return pl.pallas_call(
      _proj_body,
      grid=(N // RB,),
      in_specs=[
          pl.BlockSpec((RB, H_DIM), lambda i: (i, 0)),
          pl.BlockSpec((H_DIM, H_DIM), lambda i: (0, 0)),
          pl.BlockSpec((H_DIM, 32), lambda i: (0, 0)),
      ],
      out_specs=[
          pl.BlockSpec((RB, H_DIM), lambda i: (i, 0)),
          pl.BlockSpec((RB, 16), lambda i: (i, 0)),
          pl.BlockSpec((RB, 16), lambda i: (i, 0)),
          pl.BlockSpec((1, 32), lambda i: (0, 0)),
      ],
      out_shape=[
          jax.ShapeDtypeStruct((N, H_DIM), jnp.float32),
          jax.ShapeDtypeStruct((N, 16), jnp.float32),
          jax.ShapeDtypeStruct((N, 16), jnp.float32),
          jax.ShapeDtypeStruct((1, 32), jnp.float32),
      ],
  )(x, w, amat)


def _tc_post(acc, den, res, bias, g, be):
  return pl.pallas_call(
      _post_body,
      grid=(N // RB,),
      in_specs=[
          pl.BlockSpec((NC, RB, H_DIM), lambda i: (0, i, 0)),
          pl.BlockSpec((NC, RB, 16), lambda i: (0, i, 0)),
          pl.BlockSpec((RB, H_DIM), lambda i: (i, 0)),
          pl.BlockSpec((1, H_DIM), lambda i: (0, 0)),
          pl.BlockSpec((1, H_DIM), lambda i: (0, 0)),
          pl.BlockSpec((1, H_DIM), lambda i: (0, 0)),
      ],
      out_specs=pl.BlockSpec((RB, H_DIM), lambda i: (i, 0)),
      out_shape=jax.ShapeDtypeStruct((N, H_DIM), jnp.float32),
  )(acc, den, res, bias.reshape(1, H_DIM), g.reshape(1, H_DIM),
    be.reshape(1, H_DIM))


def _edge_body(src_hbm, dst_hbm, asrc_hbm, adst_hbm, hh_hbm, mraw_hbm,
               acc_hbm, den_hbm,
               acc_sh, den_sh, sidx, didx, srows, drows, wbuf,
               hrows0, hrows1, hrows2,
               sem_s, sem_d, sem_h0, sem_h1, sem_h2, sem_den,
               sem_a0, sem_a1, sem_a2,
               m16v):
  c = lax.axis_index("c")
  s = lax.axis_index("s")

  def zrow(r, _):
    for g in range(H_DIM // L):
      hrows0[r, pl.ds(g * L, L)] = jnp.zeros((L,), jnp.float32)
    wbuf[r, :] = jnp.zeros((L,), jnp.float32)
    return 0

  lax.fori_loop(0, RCH, zrow, 0)
  r0 = s * RPS
  zdescs = []
  for k in range(NRC):
    zdescs.append(pltpu.async_copy(
        hrows0.at[pl.ds(0, RCH)], acc_sh.at[pl.ds(r0 + k * RCH, RCH)],
        sem_a0))
    zdescs.append(pltpu.async_copy(
        wbuf.at[pl.ds(0, RCH)], den_sh.at[pl.ds(r0 + k * RCH, RCH)],
        sem_den))
  for d in zdescs:
    d.wait()
  pltpu.sync_copy(mraw_hbm, m16v)
  plsc.subcore_barrier()

  iota = lax.iota(jnp.int32, L)
  msum = jnp.maximum(m16v[pl.ds(0, L)] + m16v[pl.ds(L, L)], 0.0)
  m16 = msum + jnp.where(iota < HEADS, 0.0, 100.0)
  tile_row0 = (c * NS + s) * NCHUNK
  hb = (hrows0, hrows1, hrows2)
  sem_h = (sem_h0, sem_h1, sem_h2)
  sem_a = (sem_a0, sem_a1, sem_a2)

  def superchunk(sc, _):
    row0 = tile_row0 + sc * KB
    pltpu.sync_copy(src_hbm.at[pl.ds(row0, KB)], sidx)
    pltpu.sync_copy(dst_hbm.at[pl.ds(row0, KB)], didx)

    d_s = pltpu.async_copy(asrc_hbm.at[sidx.at[0]], srows, sem_s)
    d_d = pltpu.async_copy(adst_hbm.at[didx.at[0]], drows, sem_d)
    d_h = [None, None, None]
    d_h[0] = pltpu.async_copy(hh_hbm.at[sidx.at[0]], hrows0, sem_h0)
    den_desc = None
    acc_desc = [None, None, None]

    for j in range(KB):
      p = j % 3
      q = (j + 1) % 3
      d_s.wait()
      d_d.wait()
      d_h[p].wait()
      if den_desc is not None:
        den_desc.wait()
      if j + 1 < KB:
        if acc_desc[q] is not None:
          acc_desc[q].wait()
        d_h[q] = pltpu.async_copy(hh_hbm.at[sidx.at[j + 1]], hb[q], sem_h[q])
      hbp = hb[p]

      @plsc.parallel_loop(0, CH, unroll=4)
      def edge_w(e):
        sv = srows[e, :] + drows[e, :]
        wbuf[e, :] = jnp.exp(jnp.maximum(sv, 0.2 * sv) - m16)

      den_desc = pltpu.async_copy(wbuf, den_sh.at[didx.at[j]], sem_den,
                                  add=True)
      if j + 1 < KB:
        d_s = pltpu.async_copy(asrc_hbm.at[sidx.at[j + 1]], srows, sem_s)
        d_d = pltpu.async_copy(adst_hbm.at[didx.at[j + 1]], drows, sem_d)

      @plsc.parallel_loop(0, CH, unroll=4)
      def edge_scale(e):
        w16 = wbuf[e, :]
        for g in range(HEADS):
          hbp[e, pl.ds(g * D_HEAD, L)] = hbp[e, pl.ds(g * D_HEAD, L)] * w16[g]

      acc_desc[p] = pltpu.async_copy(hbp, acc_sh.at[didx.at[j]], sem_a[p],
                                     add=True)

    den_desc.wait()
    for d in acc_desc:
      if d is not None:
        d.wait()
    return 0

  lax.fori_loop(0, NSUPER, superchunk, 0)
  plsc.subcore_barrier()

  wdescs = []
  for k in range(NRC):
    rr = r0 + k * RCH
    wdescs.append(pltpu.async_copy(
        acc_sh.at[pl.ds(rr, RCH)], acc_hbm.at[c, pl.ds(rr, RCH)], sem_a0))
    wdescs.append(pltpu.async_copy(
        den_sh.at[pl.ds(rr, RCH)], den_hbm.at[c, pl.ds(rr, RCH)], sem_den))
  for d in wdescs:
    d.wait()


_edge_kernel = functools.partial(
    pl.kernel,
    out_type=[
        jax.ShapeDtypeStruct((NC, NPAD, H_DIM), jnp.float32),
        jax.ShapeDtypeStruct((NC, NPAD, 16), jnp.float32),
    ],
    mesh=plsc.VectorSubcoreMesh(
        core_axis_name="c", subcore_axis_name="s", num_cores=NC,
        num_subcores=NS),
    compiler_params=pltpu.CompilerParams(use_tc_tiling_on_sc=False),
    scratch_types=[
        pltpu.VMEM_SHARED((NPAD, H_DIM), jnp.float32),
        pltpu.VMEM_SHARED((NPAD, 16), jnp.float32),
        pltpu.VMEM((KB, CH), jnp.int32),
        pltpu.VMEM((KB, CH), jnp.int32),
        pltpu.VMEM((CH, 16), jnp.float32),
        pltpu.VMEM((CH, 16), jnp.float32),
        pltpu.VMEM((CH, 16), jnp.float32),
        pltpu.VMEM((CH, H_DIM), jnp.float32),
        pltpu.VMEM((CH, H_DIM), jnp.float32),
        pltpu.VMEM((CH, H_DIM), jnp.float32),
        pltpu.SemaphoreType.DMA,
        pltpu.SemaphoreType.DMA,
        pltpu.SemaphoreType.DMA,
        pltpu.SemaphoreType.DMA,
        pltpu.SemaphoreType.DMA,
        pltpu.SemaphoreType.DMA,
        pltpu.SemaphoreType.DMA,
        pltpu.SemaphoreType.DMA,
        pltpu.SemaphoreType.DMA,
        pltpu.VMEM((32,), jnp.float32),
    ],
)(_edge_body)


def _sc_edge_pass(src2d, dst2d, asrc16, adst16, hh, mraw):
  return _edge_kernel(src2d, dst2d, asrc16, adst16, hh, mraw.reshape(32))


def _build_amat(a_s, a_d):
  rows = jnp.arange(H_DIM)
  amat = jnp.zeros((H_DIM, 32), jnp.float32)
  amat = amat.at[rows, rows // D_HEAD].set(a_s.reshape(-1))
  amat = amat.at[rows, 16 + rows // D_HEAD].set(a_d.reshape(-1))
  return amat


@jax.jit
def kernel(node_features, edge_index, W_in, b_in, W1, a_src1, a_dst1, b1,
           g1, be1, W2, a_src2, a_dst2, b2, g2, be2):
  src2d = edge_index[0].reshape(E // CH, CH)
  dst2d = edge_index[1].reshape(E // CH, CH)

  x = _tc_pre(node_features, W_in, b_in)

  hh1, as1, ad1, mraw1 = _tc_proj(x, W1, _build_amat(a_src1, a_dst1))
  acc1, den1 = _sc_edge_pass(src2d, dst2d, as1, ad1, hh1, mraw1)
  x2 = _tc_post(acc1, den1, x, b1, g1, be1)

  hh2, as2, ad2, mraw2 = _tc_proj(x2, W2, _build_amat(a_src2, a_dst2))
  acc2, den2 = _sc_edge_pass(src2d, dst2d, as2, ad2, hh2, mraw2)
  return _tc_post(acc2, den2, x2, b2, g2, be2)


# speedup vs baseline: 174.2854x; 1.0464x over previous
import functools

import jax
import jax.numpy as jnp
from jax import lax
from jax.experimental import pallas as pl
from jax.experimental.pallas import tpu as pltpu
from jax.experimental.pallas import tpu_sc as plsc

N = 10000
E = 320000
F_IN = 128
H_DIM = 128
HEADS = 8
D_HEAD = 16

NC = 2
NS = 16
L = 16

RB = 2000
CH = 80
EPT = E // (NC * NS)
NCHUNK = EPT // CH
KB = 25
NSUPER = NCHUNK // KB
NPAD = 10080
RPS = NPAD // NS
RCH = 63
NRC = RPS // RCH


def _sel_matrix():
  r = lax.broadcasted_iota(jnp.int32, (H_DIM, L), 0) // D_HEAD
  cc = lax.broadcasted_iota(jnp.int32, (H_DIM, L), 1)
  return ((r == cc) & (cc < HEADS)).astype(jnp.float32)


def _proj_store(i, h, asf_ref, adf_ref, h_ref, asrc_ref, adst_ref, m_ref):
  h_ref[...] = h
  selT = _sel_matrix()
  asrc = jnp.dot(h * asf_ref[...], selT, preferred_element_type=jnp.float32)
  adst = jnp.dot(h * adf_ref[...], selT, preferred_element_type=jnp.float32)
  asrc_ref[...] = asrc
  adst_ref[...] = adst
  bm_s = jnp.max(asrc, axis=0, keepdims=True)
  bm_d = jnp.max(adst, axis=0, keepdims=True)
  bm = jnp.concatenate([bm_s, bm_d], axis=0)

  @pl.when(i == 0)
  def _():
    m_ref[...] = jnp.full_like(m_ref, -1e30)

  m_ref[...] = jnp.maximum(m_ref[...], bm)


def _preproj_body(nf_ref, win_ref, bin_ref, w_ref, asf_ref, adf_ref,
                  x_ref, h_ref, asrc_ref, adst_ref, m_ref):
  i = pl.program_id(0)
  x = (jnp.dot(nf_ref[...], win_ref[...], preferred_element_type=jnp.float32)
       + bin_ref[...])
  x_ref[...] = x
  h = jnp.dot(x, w_ref[...], preferred_element_type=jnp.float32)
  _proj_store(i, h, asf_ref, adf_ref, h_ref, asrc_ref, adst_ref, m_ref)


def _post_vals(acc_ref, den_ref, res_ref, bias_ref, g_ref, be_ref):
  a = acc_ref[0] + acc_ref[1]
  d8 = (den_ref[0] + den_ref[1])[:, 0:HEADS]
  col = lax.broadcasted_iota(jnp.int32, (HEADS, H_DIM), 1)
  row = lax.broadcasted_iota(jnp.int32, (HEADS, H_DIM), 0)
  sel = (col // D_HEAD == row).astype(jnp.float32)
  d_exp = jnp.dot(d8, sel, preferred_element_type=jnp.float32)
  out = a / jnp.maximum(d_exp, 1e-30) + bias_ref[...]
  out = jnp.where(out > 0, out, jnp.exp(jnp.minimum(out, 0.0)) - 1.0)
  xn = out + res_ref[...]
  mu = jnp.mean(xn, axis=1, keepdims=True)
  var = jnp.mean((xn - mu) * (xn - mu), axis=1, keepdims=True)
  return (xn - mu) * lax.rsqrt(var + 1e-5) * g_ref[...] + be_ref[...]


def _postproj_body(acc_ref, den_ref, res_ref, bias_ref, g_ref, be_ref,
                   w_ref, asf_ref, adf_ref,
                   x_ref, h_ref, asrc_ref, adst_ref, m_ref):
  i = pl.program_id(0)
  x = _post_vals(acc_ref, den_ref, res_ref, bias_ref, g_ref, be_ref)
  x_ref[...] = x
  h = jnp.dot(x, w_ref[...], preferred_element_type=jnp.float32)
  _proj_store(i, h, asf_ref, adf_ref, h_ref, asrc_ref, adst_ref, m_ref)


def _post_body(acc_ref, den_ref, res_ref, bias_ref, g_ref, be_ref, x_ref):
  x_ref[...] = _post_vals(acc_ref, den_ref, res_ref, bias_ref, g_ref, be_ref)


_ROW = lambda i: (i, 0)
_FIX = lambda i: (0, 0)
_B128 = pl.BlockSpec((1, H_DIM), _FIX)
_BROW = pl.BlockSpec((RB, H_DIM), _ROW)
_B16 = pl.BlockSpec((RB, 16), _ROW)
_PROJ_OUT_SPECS = [_BROW, _BROW, _B16, _B16, pl.BlockSpec((2, 16), _FIX)]
_PROJ_OUT_SHAPE = [
    jax.ShapeDtypeStruct((N, H_DIM), jnp.float32),
    jax.ShapeDtypeStruct((N, H_DIM), jnp.float32),
    jax.ShapeDtypeStruct((N, 16), jnp.float32),
    jax.ShapeDtypeStruct((N, 16), jnp.float32),
    jax.ShapeDtypeStruct((2, 16), jnp.float32),
]
_ACC_SPECS = [
    pl.BlockSpec((NC, RB, H_DIM), lambda i: (0, i, 0)),
    pl.BlockSpec((NC, RB, 16), lambda i: (0, i, 0)),
]


def _tc_preproj(nf, w_in, b_in, w, a_s, a_d):
  return pl.pallas_call(
      _preproj_body,
      grid=(N // RB,),
      in_specs=[
          pl.BlockSpec((RB, F_IN), _ROW),
          pl.BlockSpec((F_IN, H_DIM), _FIX),
          _B128,
          pl.BlockSpec((H_DIM, H_DIM), _FIX),
          _B128,
          _B128,
      ],
      out_specs=_PROJ_OUT_SPECS,
      out_shape=_PROJ_OUT_SHAPE,
  )(nf, w_in, b_in.reshape(1, H_DIM), w, a_s.reshape(1, H_DIM),
    a_d.reshape(1, H_DIM))


def _tc_postproj(acc, den, res, bias, g, be, w, a_s, a_d):
  return pl.pallas_call(
      _postproj_body,
      grid=(N // RB,),
      in_specs=_ACC_SPECS + [
          _BROW, _B128, _B128, _B128,
          pl.BlockSpec((H_DIM, H_DIM), _FIX),
          _B128,
          _B128,
      ],
      out_specs=_PROJ_OUT_SPECS,
      out_shape=_PROJ_OUT_SHAPE,
  )(acc, den, res, bias.reshape(1, H_DIM), g.reshape(1, H_DIM),
    be.reshape(1, H_DIM), w, a_s.reshape(1, H_DIM), a_d.reshape(1, H_DIM))


def _tc_post(acc, den, res, bias, g, be):
  return pl.pallas_call(
      _post_body,
      grid=(N // RB,),
      in_specs=_ACC_SPECS + [_BROW, _B128, _B128, _B128],
      out_specs=_BROW,
      out_shape=jax.ShapeDtypeStruct((N, H_DIM), jnp.float32),
  )(acc, den, res, bias.reshape(1, H_DIM), g.reshape(1, H_DIM),
    be.reshape(1, H_DIM))


def _edge_body(src_hbm, dst_hbm, asrc_hbm, adst_hbm, hh_hbm, mraw_hbm,
               acc_hbm, den_hbm,
               acc_sh, den_sh, sidx, didx, srows, drows, wbuf,
               hrows0, hrows1, hrows2,
               sem_s, sem_d, sem_h0, sem_h1, sem_h2, sem_den,
               sem_a0, sem_a1, sem_a2,
               m16v):
  c = lax.axis_index("c")
  s = lax.axis_index("s")

  def zrow(r, _):
    for g in range(H_DIM // L):
      hrows0[r, pl.ds(g * L, L)] = jnp.zeros((L,), jnp.float32)
    wbuf[r, :] = jnp.zeros((L,), jnp.float32)
    return 0

  lax.fori_loop(0, RCH, zrow, 0)
  r0 = s * RPS
  zdescs = []
  for k in range(NRC):
    zdescs.append(pltpu.async_copy(
        hrows0.at[pl.ds(0, RCH)], acc_sh.at[pl.ds(r0 + k * RCH, RCH)],
        sem_a0))
    zdescs.append(pltpu.async_copy(
        wbuf.at[pl.ds(0, RCH)], den_sh.at[pl.ds(r0 + k * RCH, RCH)],
        sem_den))
  for d in zdescs:
    d.wait()
  pltpu.sync_copy(mraw_hbm, m16v)
  plsc.subcore_barrier()

  iota = lax.iota(jnp.int32, L)
  msum = jnp.maximum(m16v[pl.ds(0, L)] + m16v[pl.ds(L, L)], 0.0)
  m16 = msum + jnp.where(iota < HEADS, 0.0, 100.0)
  tile_row0 = (c * NS + s) * NCHUNK
  hb = (hrows0, hrows1, hrows2)
  sem_h = (sem_h0, sem_h1, sem_h2)
  sem_a = (sem_a0, sem_a1, sem_a2)

  def superchunk(sc, _):
    row0 = tile_row0 + sc * KB
    pltpu.sync_copy(src_hbm.at[pl.ds(row0, KB)], sidx)
    pltpu.sync_copy(dst_hbm.at[pl.ds(row0, KB)], didx)

    d_s = pltpu.async_copy(asrc_hbm.at[sidx.at[0]], srows, sem_s)
    d_d = pltpu.async_copy(adst_hbm.at[didx.at[0]], drows, sem_d)
    d_h = [None, None, None]
    d_h[0] = pltpu.async_copy(hh_hbm.at[sidx.at[0]], hrows0, sem_h0)
    den_desc = None
    acc_desc = [None, None, None]

    for j in range(KB):
      p = j % 3
      q = (j + 1) % 3
      d_s.wait()
      d_d.wait()
      d_h[p].wait()
      if den_desc is not None:
        den_desc.wait()
      if j + 1 < KB:
        if acc_desc[q] is not None:
          acc_desc[q].wait()
        d_h[q] = pltpu.async_copy(hh_hbm.at[sidx.at[j + 1]], hb[q], sem_h[q])
      hbp = hb[p]

      @plsc.parallel_loop(0, CH, unroll=4)
      def edge_w(e):
        sv = srows[e, :] + drows[e, :]
        wbuf[e, :] = jnp.exp(jnp.maximum(sv, 0.2 * sv) - m16)

      den_desc = pltpu.async_copy(wbuf, den_sh.at[didx.at[j]], sem_den,
                                  add=True)
      if j + 1 < KB:
        d_s = pltpu.async_copy(asrc_hbm.at[sidx.at[j + 1]], srows, sem_s)
        d_d = pltpu.async_copy(adst_hbm.at[didx.at[j + 1]], drows, sem_d)

      @plsc.parallel_loop(0, CH, unroll=4)
      def edge_scale(e):
        w16 = wbuf[e, :]
        for g in range(HEADS):
          hbp[e, pl.ds(g * D_HEAD, L)] = hbp[e, pl.ds(g * D_HEAD, L)] * w16[g]

      acc_desc[p] = pltpu.async_copy(hbp, acc_sh.at[didx.at[j]], sem_a[p],
                                     add=True)

    den_desc.wait()
    for d in acc_desc:
      if d is not None:
        d.wait()
    return 0

  lax.fori_loop(0, NSUPER, superchunk, 0)
  plsc.subcore_barrier()

  wdescs = []
  for k in range(NRC):
    rr = r0 + k * RCH
    wdescs.append(pltpu.async_copy(
        acc_sh.at[pl.ds(rr, RCH)], acc_hbm.at[c, pl.ds(rr, RCH)], sem_a0))
    wdescs.append(pltpu.async_copy(
        den_sh.at[pl.ds(rr, RCH)], den_hbm.at[c, pl.ds(rr, RCH)], sem_den))
  for d in wdescs:
    d.wait()


_edge_kernel = functools.partial(
    pl.kernel,
    out_type=[
        jax.ShapeDtypeStruct((NC, NPAD, H_DIM), jnp.float32),
        jax.ShapeDtypeStruct((NC, NPAD, 16), jnp.float32),
    ],
    mesh=plsc.VectorSubcoreMesh(
        core_axis_name="c", subcore_axis_name="s", num_cores=NC,
        num_subcores=NS),
    compiler_params=pltpu.CompilerParams(use_tc_tiling_on_sc=False),
    scratch_types=[
        pltpu.VMEM_SHARED((NPAD, H_DIM), jnp.float32),
        pltpu.VMEM_SHARED((NPAD, 16), jnp.float32),
        pltpu.VMEM((KB, CH), jnp.int32),
        pltpu.VMEM((KB, CH), jnp.int32),
        pltpu.VMEM((CH, 16), jnp.float32),
        pltpu.VMEM((CH, 16), jnp.float32),
        pltpu.VMEM((CH, 16), jnp.float32),
        pltpu.VMEM((CH, H_DIM), jnp.float32),
        pltpu.VMEM((CH, H_DIM), jnp.float32),
        pltpu.VMEM((CH, H_DIM), jnp.float32),
        pltpu.SemaphoreType.DMA,
        pltpu.SemaphoreType.DMA,
        pltpu.SemaphoreType.DMA,
        pltpu.SemaphoreType.DMA,
        pltpu.SemaphoreType.DMA,
        pltpu.SemaphoreType.DMA,
        pltpu.SemaphoreType.DMA,
        pltpu.SemaphoreType.DMA,
        pltpu.SemaphoreType.DMA,
        pltpu.VMEM((32,), jnp.float32),
    ],
)(_edge_body)


def _sc_edge_pass(src2d, dst2d, asrc16, adst16, hh, mraw):
  return _edge_kernel(src2d, dst2d, asrc16, adst16, hh, mraw.reshape(32))


@jax.jit
def kernel(node_features, edge_index, W_in, b_in, W1, a_src1, a_dst1, b1,
           g1, be1, W2, a_src2, a_dst2, b2, g2, be2):
  src2d = edge_index[0].reshape(E // CH, CH)
  dst2d = edge_index[1].reshape(E // CH, CH)

  x, hh1, as1, ad1, mraw1 = _tc_preproj(node_features, W_in, b_in, W1,
                                        a_src1, a_dst1)
  acc1, den1 = _sc_edge_pass(src2d, dst2d, as1, ad1, hh1, mraw1)
  x2, hh2, as2, ad2, mraw2 = _tc_postproj(acc1, den1, x, b1, g1, be1, W2,
                                          a_src2, a_dst2)
  acc2, den2 = _sc_edge_pass(src2d, dst2d, as2, ad2, hh2, mraw2)
  return _tc_post(acc2, den2, x2, b2, g2, be2)
